# Initial kernel scaffold; baseline (speedup 1.0000x reference)
import jax
import jax.numpy as jnp
from jax.experimental import pallas as pl


def kernel(v_pos, t_pos_idx):
    raise NotImplementedError("write your pallas kernel here")


# speedup vs baseline: 11.2847x; 11.2847x over previous
import functools

import jax
import jax.numpy as jnp
from jax import lax
from jax.experimental import pallas as pl
from jax.experimental.pallas import tpu as pltpu
from jax.experimental.pallas import tpu_sc as plsc

NV = 100000
NF = 200000
NC, NS, L = 2, 16, 16
NW = NC * NS
W = 8

FT = 6272
CH = 1568
NCH = FT // CH
NPAD = 100096
VS = NPAD // NS


def _sc_body(vpos_hbm, idx_hbm, zeros_hbm, out_hbm, acc,
             idx0_v, idx1_v, idx2_v, rows0_v, rows1_v, rows2_v, nbuf):
    cid = lax.axis_index("c")
    sid = lax.axis_index("s")
    wid = sid * NC + cid
    idx_refs = (idx0_v, idx1_v, idx2_v)
    row_refs = (rows0_v, rows1_v, rows2_v)

    pltpu.sync_copy(zeros_hbm, acc.at[pl.ds(sid * VS, VS)])

    for c in range(3):
        pltpu.sync_copy(idx_hbm.at[c, wid], idx_refs[c])

    plsc.subcore_barrier()

    lanes = lax.iota(jnp.int32, 16)
    zerof16 = jnp.zeros((16,), jnp.float32)

    @pl.loop(0, CH * W // 16)
    def _(j):
        flat = j * 16 + lanes
        plsc.store_scatter(nbuf, [flat // W, flat % W], zerof16)

    @pl.loop(0, NCH)
    def _(ci):
        for c in range(3):
            pltpu.sync_copy(
                vpos_hbm.at[idx_refs[c].at[pl.ds(ci * CH, CH)]], row_refs[c]
            )

        @pl.loop(0, CH // 16)
        def _(i):
            col = i * 16 + lanes

            def comp(c, k):
                kk = jnp.full((16,), k, jnp.int32)
                return plsc.load_gather(row_refs[c], [col, kk])

            x0, y0, z0 = comp(0, 0), comp(0, 1), comp(0, 2)
            x1, y1, z1 = comp(1, 0), comp(1, 1), comp(1, 2)
            x2, y2, z2 = comp(2, 0), comp(2, 1), comp(2, 2)
            e1x, e1y, e1z = x1 - x0, y1 - y0, z1 - z0
            e2x, e2y, e2z = x2 - x0, y2 - y0, z2 - z0
            nx = e1y * e2z - e1z * e2y
            ny = e1z * e2x - e1x * e2z
            nz = e1x * e2y - e1y * e2x

            for k, v in ((0, nx), (1, ny), (2, nz)):
                kk = jnp.full((16,), k, jnp.int32)
                plsc.store_scatter(nbuf, [col, kk], v)

        for c in range(3):
            pltpu.sync_copy(
                nbuf, acc.at[idx_refs[c].at[pl.ds(ci * CH, CH)]], add=True
            )

    plsc.subcore_barrier()

    pltpu.sync_copy(
        acc.at[pl.ds(sid * VS, VS)], out_hbm.at[cid, pl.ds(sid * VS, VS)]
    )


@functools.cache
def _sc_scatter():
    return pl.kernel(
        _sc_body,
        out_type=jax.ShapeDtypeStruct((NC, NPAD, W), jnp.float32),
        mesh=plsc.VectorSubcoreMesh(
            core_axis_name="c", subcore_axis_name="s",
            num_cores=NC, num_subcores=NS,
        ),
        scratch_types=[
            pltpu.VMEM_SHARED((NPAD, W), jnp.float32),
            pltpu.VMEM((FT,), jnp.int32),
            pltpu.VMEM((FT,), jnp.int32),
            pltpu.VMEM((FT,), jnp.int32),
            pltpu.VMEM((CH, W), jnp.float32),
            pltpu.VMEM((CH, W), jnp.float32),
            pltpu.VMEM((CH, W), jnp.float32),
            pltpu.VMEM((CH, W), jnp.float32),
        ],
        compiler_params=pltpu.CompilerParams(
            needs_layout_passes=False, use_tc_tiling_on_sc=False
        ),
    )


def _tc_normalize_body(part_ref, out_ref):
    x = part_ref[0] + part_ref[1]
    ri = lax.broadcasted_iota(jnp.int32, (128, 128), 0)
    ci = lax.broadcasted_iota(jnp.int32, (128, 128), 1)
    g = (((ri // W) == (ci // W)) & ((ri % W) < 3)).astype(jnp.float32)
    sq = jnp.dot(x * x, g, preferred_element_type=jnp.float32)
    li = lax.broadcasted_iota(jnp.int32, x.shape, 1)
    fb = ((li % W) == 2).astype(jnp.float32)
    y = jnp.where(sq > 1e-20, x, fb)
    sqy = jnp.dot(y * y, g, preferred_element_type=jnp.float32)
    inv = 1.0 / jnp.maximum(jnp.sqrt(sqy), 1e-12)
    out_ref[...] = y * inv


_tc_normalize = pl.pallas_call(
    _tc_normalize_body,
    out_shape=jax.ShapeDtypeStruct((NPAD * W // 128, 128), jnp.float32),
)


@jax.jit
def kernel(v_pos, t_pos_idx):
    vpos_pad = jnp.pad(v_pos, ((0, NPAD - NV), (0, W - 3)))
    idx = t_pos_idx.astype(jnp.int32)
    idx = jnp.pad(idx, ((0, NW * FT - NF), (0, 0)), constant_values=NV)
    idx_hbm = idx.T.reshape(3, NW, FT)
    zeros = jnp.zeros((VS, W), jnp.float32)

    partials = _sc_scatter()(vpos_pad, idx_hbm, zeros)
    out = _tc_normalize(partials.reshape(NC, NPAD * W // 128, 128))
    return out.reshape(NPAD, W)[:NV, :3]
